# baseline (speedup 1.0000x reference)
import functools

import jax
import jax.numpy as jnp
from jax import lax
from jax.experimental import pallas as pl
from jax.experimental.pallas import tpu as pltpu
from jax.experimental.pallas import tpu_sc as plsc

_N = 50000
_E = 800000
_NP = 51200
_RPT = _NP // 16
_NSS = 98
_EPT = _NSS * 512
_EP = 16 * _EPT
_BR = 2000
_NBLK = _N // _BR
_EPS = 1e-5

_mesh = plsc.VectorSubcoreMesh(core_axis_name="c", subcore_axis_name="s")
_sc_params = pltpu.CompilerParams(use_tc_tiling_on_sc=False)


def _make_agg(dc):
    out_t = [jax.ShapeDtypeStruct((_NP, dc), jnp.float32)] * 2
    scratch = [
        pltpu.VMEM((4, 128), jnp.int32),
        pltpu.VMEM((4, 128), jnp.int32),
        pltpu.VMEM((4, 128, dc), jnp.float32),
        pltpu.VMEM_SHARED((_NP, dc), jnp.float32),
        pltpu.SemaphoreType.DMA,
        pltpu.SemaphoreType.DMA,
    ]

    @functools.partial(pl.kernel, out_type=out_t, mesh=_mesh,
                       scratch_types=scratch, compiler_params=_sc_params)
    def agg(u0, u1, src4, dst4, zer, out0, out1, sbuf, dbuf, rows, acc,
            isem, gsem):
        cid = lax.axis_index("c")
        tid = lax.axis_index("s")
        pltpu.sync_copy(zer, acc.at[pl.ds(tid * _RPT, _RPT), :])
        plsc.subcore_barrier()
        for c, (u, out) in enumerate(((u0, out0), (u1, out1))):
            @pl.when(cid == c)
            def _():
                def body(j, carry):
                    r = tid * _NSS + j
                    ca = pltpu.async_copy(src4.at[r], sbuf, isem)
                    cb = pltpu.async_copy(dst4.at[r], dbuf, isem)
                    ca.wait()
                    cb.wait()
                    gs = [pltpu.async_copy(u.at[sbuf.at[k]], rows.at[k], gsem)
                          for k in range(4)]
                    for g in gs:
                        g.wait()
                    for k in range(4):
                        pltpu.sync_copy(rows.at[k], acc.at[dbuf.at[k]],
                                        add=True)
                    return carry
                lax.fori_loop(0, _NSS, body, 0)
                plsc.subcore_barrier()
                pltpu.sync_copy(acc.at[pl.ds(tid * _RPT, _RPT), :],
                                out.at[pl.ds(tid * _RPT, _RPT), :])
        return None
    return agg


@functools.partial(
    pl.kernel,
    out_type=jax.ShapeDtypeStruct((_NP,), jnp.float32),
    mesh=_mesh,
    scratch_types=[
        pltpu.VMEM((4, 128), jnp.int32),
        pltpu.VMEM((128,), jnp.float32),
        pltpu.VMEM_SHARED((_NP,), jnp.float32),
        pltpu.SemaphoreType.DMA,
    ],
    compiler_params=_sc_params)
def _deg_kernel(dst4, zer1, ones_h, dego, dbuf, onesv, acc, isem):
    cid = lax.axis_index("c")
    tid = lax.axis_index("s")

    @pl.when(cid == 0)
    def _():
        pltpu.sync_copy(ones_h, onesv)
        pltpu.sync_copy(zer1, acc.at[pl.ds(tid * _RPT, _RPT)])
        plsc.subcore_barrier()

        def body(j, carry):
            r = tid * _NSS + j
            pltpu.async_copy(dst4.at[r], dbuf, isem).wait()
            for k in range(4):
                pltpu.sync_copy(onesv, acc.at[dbuf.at[k]], add=True)
            return carry
        lax.fori_loop(0, _NSS, body, 0)
        plsc.subcore_barrier()
        pltpu.sync_copy(acc.at[pl.ds(tid * _RPT, _RPT)],
                        dego.at[pl.ds(tid * _RPT, _RPT)])


_agg32 = _make_agg(32)
_agg16 = _make_agg(16)


def _row_spec(w):
    return pl.BlockSpec((_BR, w), lambda i: (i, 0))


def _const_spec(shape):
    return pl.BlockSpec(shape, lambda i: (0, 0))


def _tc_prep(deg2, x64, W1p):
    def body(deg_r, x_r, w_r, dv_r, ul_r, uh_r):
        dv = lax.rsqrt(deg_r[...] + 1.0)
        t = lax.dot_general(x_r[...], w_r[...], (((1,), (0,)), ((), ())),
                            preferred_element_type=jnp.float32)
        u = t * dv
        dv_r[...] = dv
        ul_r[...] = u[:, :32]
        uh_r[...] = u[:, 32:]
    return pl.pallas_call(
        body, grid=(_NBLK,),
        in_specs=[_row_spec(1), _row_spec(64), _const_spec(W1p.shape)],
        out_specs=[_row_spec(1), _row_spec(32), _row_spec(32)],
        out_shape=[jax.ShapeDtypeStruct((_N, 1), jnp.float32),
                   jax.ShapeDtypeStruct((_N, 32), jnp.float32),
                   jax.ShapeDtypeStruct((_N, 32), jnp.float32)],
    )(deg2, x64, W1p)


def _tc_layer(S0, S1, u0, u1, dinv, W, b1r):
    dh = S0.shape[1]
    dout = W.shape[1] if W is not None else 2 * dh

    def body(s0, s1, u0r, u1r, dv, *rest):
        if W is not None:
            w, bb, a_r, sm_r, sq_r = rest
        else:
            bb, a_r, sm_r, sq_r = rest
        i = pl.program_id(0)
        y = jnp.concatenate([s0[...] + u0r[...], s1[...] + u1r[...]],
                            axis=1) * dv[...]
        if W is not None:
            z = lax.dot_general(y, w[...], (((1,), (0,)), ((), ())),
                                preferred_element_type=jnp.float32,
                                precision=lax.Precision.HIGHEST)
        else:
            z = y
        a = jnp.maximum(z + bb[...], 0.0)
        a_r[...] = a

        @pl.when(i == 0)
        def _():
            sm_r[...] = jnp.zeros_like(sm_r)
            sq_r[...] = jnp.zeros_like(sq_r)
        sm_r[...] += jnp.sum(a, axis=0, keepdims=True)
        sq_r[...] += jnp.sum(a * a, axis=0, keepdims=True)

    in_specs = [_row_spec(dh), _row_spec(dh), _row_spec(dh), _row_spec(dh),
                _row_spec(1)]
    args = [S0, S1, u0, u1, dinv]
    if W is not None:
        in_specs.append(_const_spec(W.shape))
        args.append(W)
    in_specs.append(_const_spec((1, dout)))
    args.append(b1r)
    return pl.pallas_call(
        body, grid=(_NBLK,),
        in_specs=in_specs,
        out_specs=[_row_spec(dout), _const_spec((1, dout)),
                   _const_spec((1, dout))],
        out_shape=[jax.ShapeDtypeStruct((_N, dout), jnp.float32),
                   jax.ShapeDtypeStruct((1, dout), jnp.float32),
                   jax.ShapeDtypeStruct((1, dout), jnp.float32)],
    )(*args)


def _tc_bn_next(a, sm, sq, g1r, be1r, dinv, W):
    d = a.shape[1]
    dn = W.shape[1] if W is not None else d
    half = dn // 2

    def body(a_r, sm_r, sq_r, g_r, be_r, dv, *rest):
        if W is not None:
            w, ul_r, uh_r = rest
        else:
            ul_r, uh_r = rest
        mu = sm_r[...] / _N
        var = sq_r[...] / _N - mu * mu
        sc = g_r[...] * lax.rsqrt(var + _EPS)
        h = (a_r[...] - mu) * sc + be_r[...]
        if W is not None:
            t = lax.dot_general(h, w[...], (((1,), (0,)), ((), ())),
                                preferred_element_type=jnp.float32)
        else:
            t = h
        u = t * dv[...]
        ul_r[...] = u[:, :half]
        uh_r[...] = u[:, half:]

    in_specs = [_row_spec(d), _const_spec((1, d)), _const_spec((1, d)),
                _const_spec((1, d)), _const_spec((1, d)), _row_spec(1)]
    args = [a, sm, sq, g1r, be1r, dinv]
    if W is not None:
        in_specs.append(_const_spec(W.shape))
        args.append(W)
    return pl.pallas_call(
        body, grid=(_NBLK,),
        in_specs=in_specs,
        out_specs=[_row_spec(half), _row_spec(half)],
        out_shape=[jax.ShapeDtypeStruct((_N, half), jnp.float32),
                   jax.ShapeDtypeStruct((_N, half), jnp.float32)],
    )(*args)


def _tc_final(a, sm, sq, g1r, be1r, Wc, bc1r):
    d = a.shape[1]
    nc = Wc.shape[1]

    def body(a_r, sm_r, sq_r, g_r, be_r, w, bb, out_r, h_r):
        mu = sm_r[...] / _N
        var = sq_r[...] / _N - mu * mu
        sc = g_r[...] * lax.rsqrt(var + _EPS)
        h = (a_r[...] - mu) * sc + be_r[...]
        h_r[...] = h
        out_r[...] = lax.dot_general(h, w[...], (((1,), (0,)), ((), ())),
                                     preferred_element_type=jnp.float32) + bb[...]

    return pl.pallas_call(
        body, grid=(_NBLK,),
        in_specs=[_row_spec(d), _const_spec((1, d)), _const_spec((1, d)),
                  _const_spec((1, d)), _const_spec((1, d)),
                  _const_spec(Wc.shape), _const_spec((1, nc))],
        out_specs=[_row_spec(nc), _row_spec(d)],
        out_shape=[jax.ShapeDtypeStruct((_N, nc), jnp.float32),
                   jax.ShapeDtypeStruct((_N, d), jnp.float32)],
    )(a, sm, sq, g1r, be1r, Wc, bc1r)


def kernel(x, edge_index, W1, b1, W2, b2, W3, b3, W4, b4,
           g1, be1, g2, be2, g3, be3, g4, be4, Wc, bc):
    f32 = jnp.float32
    src = edge_index[0]
    dst = edge_index[1]
    pad = _EP - _E
    src4 = jnp.concatenate([src, jnp.zeros((pad,), src.dtype)]).reshape(-1, 4, 128)
    dst4 = jnp.concatenate([dst, jnp.full((pad,), _N, dst.dtype)]).reshape(-1, 4, 128)
    zer32 = jnp.zeros((_RPT, 32), f32)
    zer16 = jnp.zeros((_RPT, 16), f32)
    zer1 = jnp.zeros((_RPT,), f32)
    ones_h = jnp.ones((128,), f32)

    deg = _deg_kernel(dst4, zer1, ones_h)
    deg2 = deg[:_N].reshape(_N, 1)
    x64 = jnp.pad(x, ((0, 0), (0, 64 - x.shape[1])))
    W1p = jnp.pad(W1, ((0, 64 - W1.shape[0]), (0, 0)))
    dinv, u0, u1 = _tc_prep(deg2, x64, W1p)

    S0, S1 = _agg32(u0, u1, src4, dst4, zer32)
    a1, sm1, sq1 = _tc_layer(S0, S1, u0, u1, dinv, None, b1.reshape(1, -1))
    u0, u1 = _tc_bn_next(a1, sm1, sq1, g1.reshape(1, -1), be1.reshape(1, -1),
                         dinv, None)
    S0, S1 = _agg32(u0, u1, src4, dst4, zer32)
    a2, sm2, sq2 = _tc_layer(S0, S1, u0, u1, dinv, W2, b2.reshape(1, -1))
    u0, u1 = _tc_bn_next(a2, sm2, sq2, g2.reshape(1, -1), be2.reshape(1, -1),
                         dinv, W3)
    S0, S1 = _agg32(u0, u1, src4, dst4, zer32)
    a3, sm3, sq3 = _tc_layer(S0, S1, u0, u1, dinv, None, b3.reshape(1, -1))
    u0, u1 = _tc_bn_next(a3, sm3, sq3, g3.reshape(1, -1), be3.reshape(1, -1),
                         dinv, W4)
    S0, S1 = _agg16(u0, u1, src4, dst4, zer16)
    a4, sm4, sq4 = _tc_layer(S0, S1, u0, u1, dinv, None, b4.reshape(1, -1))
    out, h4 = _tc_final(a4, sm4, sq4, g4.reshape(1, -1), be4.reshape(1, -1),
                        Wc, bc.reshape(1, -1))
    return (out, h4)


# speedup vs baseline: 1.0432x; 1.0432x over previous
import functools

import jax
import jax.numpy as jnp
from jax import lax
from jax.experimental import pallas as pl
from jax.experimental.pallas import tpu as pltpu
from jax.experimental.pallas import tpu_sc as plsc

_N = 50000
_E = 800000
_NP = 51200
_RPT = _NP // 16
_NSS = 98
_EPT = _NSS * 512
_EP = 16 * _EPT
_BR = 2000
_NBLK = _N // _BR
_EPS = 1e-5

_mesh = plsc.VectorSubcoreMesh(core_axis_name="c", subcore_axis_name="s")
_sc_params = pltpu.CompilerParams(use_tc_tiling_on_sc=False)


def _make_agg(dc):
    out_t = [jax.ShapeDtypeStruct((_NP, dc), jnp.float32)] * 2
    scratch = [
        pltpu.VMEM((512,), jnp.int32),
        pltpu.VMEM((512,), jnp.int32),
        pltpu.VMEM((512, dc), jnp.float32),
        pltpu.VMEM_SHARED((_NP, dc), jnp.float32),
        pltpu.SemaphoreType.DMA,
        pltpu.SemaphoreType.DMA,
    ]

    @functools.partial(pl.kernel, out_type=out_t, mesh=_mesh,
                       scratch_types=scratch, compiler_params=_sc_params)
    def agg(u0, u1, src4, dst4, zer, out0, out1, sbuf, dbuf, rows, acc,
            isem, gsem):
        cid = lax.axis_index("c")
        tid = lax.axis_index("s")
        pltpu.sync_copy(zer, acc.at[pl.ds(tid * _RPT, _RPT), :])
        plsc.subcore_barrier()
        for c, (u, out) in enumerate(((u0, out0), (u1, out1))):
            @pl.when(cid == c)
            def _():
                def body(j, carry):
                    r = tid * _NSS + j
                    ca = pltpu.async_copy(src4.at[r], sbuf, isem)
                    cb = pltpu.async_copy(dst4.at[r], dbuf, isem)
                    ca.wait()
                    cb.wait()
                    pltpu.async_copy(u.at[sbuf], rows, gsem).wait()
                    pltpu.sync_copy(rows, acc.at[dbuf], add=True)
                    return carry
                lax.fori_loop(0, _NSS, body, 0)
                plsc.subcore_barrier()
                pltpu.sync_copy(acc.at[pl.ds(tid * _RPT, _RPT), :],
                                out.at[pl.ds(tid * _RPT, _RPT), :])
        return None
    return agg


@functools.partial(
    pl.kernel,
    out_type=jax.ShapeDtypeStruct((_NP,), jnp.float32),
    mesh=_mesh,
    scratch_types=[
        pltpu.VMEM((512,), jnp.int32),
        pltpu.VMEM((512,), jnp.float32),
        pltpu.VMEM_SHARED((_NP,), jnp.float32),
        pltpu.SemaphoreType.DMA,
    ],
    compiler_params=_sc_params)
def _deg_kernel(dst4, zer1, ones_h, dego, dbuf, onesv, acc, isem):
    cid = lax.axis_index("c")
    tid = lax.axis_index("s")

    @pl.when(cid == 0)
    def _():
        pltpu.sync_copy(ones_h, onesv)
        pltpu.sync_copy(zer1, acc.at[pl.ds(tid * _RPT, _RPT)])
        plsc.subcore_barrier()

        def body(j, carry):
            r = tid * _NSS + j
            pltpu.async_copy(dst4.at[r], dbuf, isem).wait()
            pltpu.sync_copy(onesv, acc.at[dbuf], add=True)
            return carry
        lax.fori_loop(0, _NSS, body, 0)
        plsc.subcore_barrier()
        pltpu.sync_copy(acc.at[pl.ds(tid * _RPT, _RPT)],
                        dego.at[pl.ds(tid * _RPT, _RPT)])


_agg32 = _make_agg(32)
_agg16 = _make_agg(16)


def _row_spec(w):
    return pl.BlockSpec((_BR, w), lambda i: (i, 0))


def _const_spec(shape):
    return pl.BlockSpec(shape, lambda i: (0, 0))


def _tc_prep(deg2, x64, W1p):
    def body(deg_r, x_r, w_r, dv_r, ul_r, uh_r):
        dv = lax.rsqrt(deg_r[...] + 1.0)
        t = lax.dot_general(x_r[...], w_r[...], (((1,), (0,)), ((), ())),
                            preferred_element_type=jnp.float32)
        u = t * dv
        dv_r[...] = dv
        ul_r[...] = u[:, :32]
        uh_r[...] = u[:, 32:]
    return pl.pallas_call(
        body, grid=(_NBLK,),
        in_specs=[_row_spec(1), _row_spec(64), _const_spec(W1p.shape)],
        out_specs=[_row_spec(1), _row_spec(32), _row_spec(32)],
        out_shape=[jax.ShapeDtypeStruct((_N, 1), jnp.float32),
                   jax.ShapeDtypeStruct((_N, 32), jnp.float32),
                   jax.ShapeDtypeStruct((_N, 32), jnp.float32)],
    )(deg2, x64, W1p)


def _tc_layer(S0, S1, u0, u1, dinv, W, b1r):
    dh = S0.shape[1]
    dout = W.shape[1] if W is not None else 2 * dh

    def body(s0, s1, u0r, u1r, dv, *rest):
        if W is not None:
            w, bb, a_r, sm_r, sq_r = rest
        else:
            bb, a_r, sm_r, sq_r = rest
        i = pl.program_id(0)
        y = jnp.concatenate([s0[...] + u0r[...], s1[...] + u1r[...]],
                            axis=1) * dv[...]
        if W is not None:
            z = lax.dot_general(y, w[...], (((1,), (0,)), ((), ())),
                                preferred_element_type=jnp.float32,
                                precision=lax.Precision.HIGHEST)
        else:
            z = y
        a = jnp.maximum(z + bb[...], 0.0)
        a_r[...] = a

        @pl.when(i == 0)
        def _():
            sm_r[...] = jnp.zeros_like(sm_r)
            sq_r[...] = jnp.zeros_like(sq_r)
        sm_r[...] += jnp.sum(a, axis=0, keepdims=True)
        sq_r[...] += jnp.sum(a * a, axis=0, keepdims=True)

    in_specs = [_row_spec(dh), _row_spec(dh), _row_spec(dh), _row_spec(dh),
                _row_spec(1)]
    args = [S0, S1, u0, u1, dinv]
    if W is not None:
        in_specs.append(_const_spec(W.shape))
        args.append(W)
    in_specs.append(_const_spec((1, dout)))
    args.append(b1r)
    return pl.pallas_call(
        body, grid=(_NBLK,),
        in_specs=in_specs,
        out_specs=[_row_spec(dout), _const_spec((1, dout)),
                   _const_spec((1, dout))],
        out_shape=[jax.ShapeDtypeStruct((_N, dout), jnp.float32),
                   jax.ShapeDtypeStruct((1, dout), jnp.float32),
                   jax.ShapeDtypeStruct((1, dout), jnp.float32)],
    )(*args)


def _tc_bn_next(a, sm, sq, g1r, be1r, dinv, W):
    d = a.shape[1]
    dn = W.shape[1] if W is not None else d
    half = dn // 2

    def body(a_r, sm_r, sq_r, g_r, be_r, dv, *rest):
        if W is not None:
            w, ul_r, uh_r = rest
        else:
            ul_r, uh_r = rest
        mu = sm_r[...] / _N
        var = sq_r[...] / _N - mu * mu
        sc = g_r[...] * lax.rsqrt(var + _EPS)
        h = (a_r[...] - mu) * sc + be_r[...]
        if W is not None:
            t = lax.dot_general(h, w[...], (((1,), (0,)), ((), ())),
                                preferred_element_type=jnp.float32)
        else:
            t = h
        u = t * dv[...]
        ul_r[...] = u[:, :half]
        uh_r[...] = u[:, half:]

    in_specs = [_row_spec(d), _const_spec((1, d)), _const_spec((1, d)),
                _const_spec((1, d)), _const_spec((1, d)), _row_spec(1)]
    args = [a, sm, sq, g1r, be1r, dinv]
    if W is not None:
        in_specs.append(_const_spec(W.shape))
        args.append(W)
    return pl.pallas_call(
        body, grid=(_NBLK,),
        in_specs=in_specs,
        out_specs=[_row_spec(half), _row_spec(half)],
        out_shape=[jax.ShapeDtypeStruct((_N, half), jnp.float32),
                   jax.ShapeDtypeStruct((_N, half), jnp.float32)],
    )(*args)


def _tc_final(a, sm, sq, g1r, be1r, Wc, bc1r):
    d = a.shape[1]
    nc = Wc.shape[1]

    def body(a_r, sm_r, sq_r, g_r, be_r, w, bb, out_r, h_r):
        mu = sm_r[...] / _N
        var = sq_r[...] / _N - mu * mu
        sc = g_r[...] * lax.rsqrt(var + _EPS)
        h = (a_r[...] - mu) * sc + be_r[...]
        h_r[...] = h
        out_r[...] = lax.dot_general(h, w[...], (((1,), (0,)), ((), ())),
                                     preferred_element_type=jnp.float32) + bb[...]

    return pl.pallas_call(
        body, grid=(_NBLK,),
        in_specs=[_row_spec(d), _const_spec((1, d)), _const_spec((1, d)),
                  _const_spec((1, d)), _const_spec((1, d)),
                  _const_spec(Wc.shape), _const_spec((1, nc))],
        out_specs=[_row_spec(nc), _row_spec(d)],
        out_shape=[jax.ShapeDtypeStruct((_N, nc), jnp.float32),
                   jax.ShapeDtypeStruct((_N, d), jnp.float32)],
    )(a, sm, sq, g1r, be1r, Wc, bc1r)


def kernel(x, edge_index, W1, b1, W2, b2, W3, b3, W4, b4,
           g1, be1, g2, be2, g3, be3, g4, be4, Wc, bc):
    f32 = jnp.float32
    src = edge_index[0]
    dst = edge_index[1]
    pad = _EP - _E
    src4 = jnp.concatenate([src, jnp.zeros((pad,), src.dtype)]).reshape(-1, 512)
    dst4 = jnp.concatenate([dst, jnp.full((pad,), _N, dst.dtype)]).reshape(-1, 512)
    zer32 = jnp.zeros((_RPT, 32), f32)
    zer16 = jnp.zeros((_RPT, 16), f32)
    zer1 = jnp.zeros((_RPT,), f32)
    ones_h = jnp.ones((512,), f32)

    deg = _deg_kernel(dst4, zer1, ones_h)
    deg2 = deg[:_N].reshape(_N, 1)
    x64 = jnp.pad(x, ((0, 0), (0, 64 - x.shape[1])))
    W1p = jnp.pad(W1, ((0, 64 - W1.shape[0]), (0, 0)))
    dinv, u0, u1 = _tc_prep(deg2, x64, W1p)

    S0, S1 = _agg32(u0, u1, src4, dst4, zer32)
    a1, sm1, sq1 = _tc_layer(S0, S1, u0, u1, dinv, None, b1.reshape(1, -1))
    u0, u1 = _tc_bn_next(a1, sm1, sq1, g1.reshape(1, -1), be1.reshape(1, -1),
                         dinv, None)
    S0, S1 = _agg32(u0, u1, src4, dst4, zer32)
    a2, sm2, sq2 = _tc_layer(S0, S1, u0, u1, dinv, W2, b2.reshape(1, -1))
    u0, u1 = _tc_bn_next(a2, sm2, sq2, g2.reshape(1, -1), be2.reshape(1, -1),
                         dinv, W3)
    S0, S1 = _agg32(u0, u1, src4, dst4, zer32)
    a3, sm3, sq3 = _tc_layer(S0, S1, u0, u1, dinv, None, b3.reshape(1, -1))
    u0, u1 = _tc_bn_next(a3, sm3, sq3, g3.reshape(1, -1), be3.reshape(1, -1),
                         dinv, W4)
    S0, S1 = _agg16(u0, u1, src4, dst4, zer16)
    a4, sm4, sq4 = _tc_layer(S0, S1, u0, u1, dinv, None, b4.reshape(1, -1))
    out, h4 = _tc_final(a4, sm4, sq4, g4.reshape(1, -1), be4.reshape(1, -1),
                        Wc, bc.reshape(1, -1))
    return (out, h4)
